# baseline (speedup 1.0000x reference)
import functools

import jax
import jax.numpy as jnp
from jax import lax
from jax.experimental import pallas as pl
from jax.experimental.pallas import tpu as pltpu
from jax.experimental.pallas import tpu_sc as plsc

_N = 10000
_E = 320000
_NPAD = 10112
_TRASH = _N
_NC, _NS = 2, 16
_NTILES = _NC * _NS
_BE = 224
_BH = 112
_NB = 2
_ETOT = _E + _N
_KCH = _NB * (-(-_ETOT // (_NTILES * _BE * _NB)))
_EPAD = _NTILES * _BE * _KCH
_EALLOC = _EPAD + _NB * _BE
_BN = 1000

_f32 = jnp.float32
_i32 = jnp.int32

_GDN = lax.GatherDimensionNumbers(
    offset_dims=(), collapsed_slice_dims=(0,), start_index_map=(0,))


def _vgather(vec, idx):
    return lax.gather(vec, idx[:, None], _GDN, (1,),
                      mode=lax.GatherScatterMode.PROMISE_IN_BOUNDS)


def _make_edge_kernel(srow, excol, nck, per_head):
    mesh = plsc.VectorSubcoreMesh(
        core_axis_name="c", subcore_axis_name="s",
        num_cores=_NC, num_subcores=_NS)
    rows_sub = _NPAD // _NS

    @functools.partial(
        pl.kernel,
        out_type=jax.ShapeDtypeStruct((_NC, _NPAD, srow), _f32),
        mesh=mesh,
        compiler_params=pltpu.CompilerParams(
            use_tc_tiling_on_sc=False, needs_layout_passes=False),
        scratch_types=[
            [pltpu.VMEM((_BE,), _i32)] * _NB,
            [pltpu.VMEM((_BE,), _i32)] * _NB,
            [[pltpu.VMEM((_BH,), _i32)] * 2] * _NB,
            [pltpu.VMEM((_BE, srow), _f32)] * _NB,
            [pltpu.VMEM((_BE, 16), _f32)] * _NB,
            [pltpu.VMEM((_BE, srow), _f32)] * _NB,
            pltpu.VMEM((16,), _f32),
            pltpu.VMEM_SHARED((_NPAD, srow), _f32),
            [pltpu.SemaphoreType.DMA] * _NB,
            [pltpu.SemaphoreType.DMA] * _NB,
            [pltpu.SemaphoreType.DMA] * _NB,
        ],
    )
    def edge_kernel(h_hbm, b_hbm, src_hbm, dst_hbm, cvec_hbm, zeros_hbm,
                    out_hbm, sidx, didx, didxs, hrows, brows, scat,
                    cvec, acc, sem_g, sem_s, sem_i):
        cid = lax.axis_index("c")
        sid = lax.axis_index("s")
        wid = cid * _NS + sid
        rb = sid * rows_sub
        pltpu.sync_copy(zeros_hbm.at[pl.ds(rb, rows_sub)],
                        acc.at[pl.ds(rb, rows_sub)])
        pltpu.sync_copy(cvec_hbm, cvec)
        plsc.subcore_barrier()
        cv = cvec[...]
        ii = lax.iota(_i32, 16)
        half = lax.shift_right_logical(ii, 3)
        scale_idx = [half + 2 * k for k in range(nck)]
        ebase = wid * (_KCH * _BE)

        def load_idx(cb, b):
            pltpu.sync_copy(src_hbm.at[pl.ds(cb, _BE)], sidx[b])
            pltpu.sync_copy(dst_hbm.at[pl.ds(cb, _BE)], didx[b])

        def _gather_parts(b):
            out = []
            for j in range(2):
                hs = pl.ds(j * _BH, _BH)
                out.append((h_hbm.at[sidx[b].at[hs]], hrows[b].at[hs]))
                out.append((b_hbm.at[didx[b].at[hs]], brows[b].at[hs]))
            return out

        def issue_gathers(b):
            for src, dst in _gather_parts(b):
                pltpu.async_copy(src, dst, sem_g[b])

        def wait_gathers(b):
            for src, dst in _gather_parts(b):
                pltpu.make_async_copy(src, dst, sem_g[b]).wait()

        def issue_scatters(b):
            for j in range(2):
                pltpu.async_copy(scat[b].at[pl.ds(j * _BH, _BH)],
                                 acc.at[didxs[b][j]], sem_s[b], add=True)

        def wait_scatter(b):
            for j in range(2):
                pltpu.make_async_copy(scat[b].at[pl.ds(j * _BH, _BH)],
                                      acc.at[didxs[b][j]], sem_s[b]).wait()

        def compute(b):
            hr = hrows[b]
            br = brows[b]
            st = scat[b]

            @plsc.parallel_loop(0, _BE, unroll=16)
            def _(e):
                z = hr[e, pl.ds(excol, 16)] + br[e, :]
                zz = jnp.where(z >= 0, z, 0.2 * z) - cv
                ex = jnp.exp(zz)
                for k2 in range(nck):
                    sc = _vgather(ex, scale_idx[k2]) if per_head else ex
                    hv = hr[e, pl.ds(k2 * 16, 16)]
                    st[e, pl.ds(k2 * 16, 16)] = hv * sc
                st[e, pl.ds(excol, 16)] = ex

        for b in range(_NB):
            load_idx(ebase + b * _BE, b)
            issue_gathers(b)

        def ringbody(go, carry):
            for b in range(_NB):
                wait_gathers(b)

                @pl.when(go > 0)
                def _():
                    wait_scatter(b)

                cb0 = ebase + (go * _NB + b) * _BE
                cp_i0 = pltpu.async_copy(dst_hbm.at[pl.ds(cb0, _BH)],
                                         didxs[b][0], sem_i[b])
                cp_i0b = pltpu.async_copy(dst_hbm.at[pl.ds(cb0 + _BH, _BH)],
                                          didxs[b][1], sem_i[b])
                cb2 = cb0 + _NB * _BE
                cp_i1 = pltpu.async_copy(src_hbm.at[pl.ds(cb2, _BE)],
                                         sidx[b], sem_i[b])
                cp_i2 = pltpu.async_copy(dst_hbm.at[pl.ds(cb2, _BE)],
                                         didx[b], sem_i[b])
                compute(b)
                cp_i0.wait()
                cp_i0b.wait()
                cp_i1.wait()
                cp_i2.wait()
                issue_scatters(b)
                issue_gathers(b)
            return carry

        lax.fori_loop(0, _KCH // _NB, ringbody, 0)
        for b in range(_NB):
            wait_gathers(b)
            wait_scatter(b)
        plsc.subcore_barrier()
        pltpu.sync_copy(acc.at[pl.ds(rb, rows_sub)],
                        out_hbm.at[cid, pl.ds(rb, rows_sub)])

    return edge_kernel


_edge1 = _make_edge_kernel(srow=80, excol=64, nck=4, per_head=True)
_edge2 = _make_edge_kernel(srow=64, excol=48, nck=3, per_head=False)


def _k1_body(x_ref, w1_ref, ms_ref, md_ref, ha_ref, b_ref, ca_ref,
             cb_ref):
    i = pl.program_id(0)
    h = jnp.dot(x_ref[...], w1_ref[...], preferred_element_type=_f32)
    a = jnp.dot(h, ms_ref[...], preferred_element_type=_f32)
    b = jnp.dot(h, md_ref[...], preferred_element_type=_f32)
    ha_ref[...] = jnp.concatenate([h, a], axis=1)
    b_ref[...] = b
    am = jnp.max(a, axis=0, keepdims=True)
    bm = jnp.max(b, axis=0, keepdims=True)

    @pl.when(i == 0)
    def _():
        ca_ref[...] = am
        cb_ref[...] = bm

    @pl.when(i != 0)
    def _():
        ca_ref[...] = jnp.maximum(ca_ref[...], am)
        cb_ref[...] = jnp.maximum(cb_ref[...], bm)


def _k2_body(acc_ref, kb_ref, bias_ref, g_ref, be_ref, w2_ref, as2_ref,
             bs2_ref, h2_ref, b2_ref, ca_ref, cb_ref):
    i = pl.program_id(0)
    acc = acc_ref[...]
    s = acc[0] + acc[1]
    num = s[:, :64]
    den = s[:, 64:72]
    denf = jnp.dot(den, kb_ref[...], preferred_element_type=_f32)
    o = num / (denf + 1e-16) + bias_ref[...]
    mu = jnp.mean(o, axis=-1, keepdims=True)
    var = jnp.mean((o - mu) ** 2, axis=-1, keepdims=True)
    o = (o - mu) / jnp.sqrt(var + 1e-5) * g_ref[...] + be_ref[...]
    o = jnp.where(o > 0, o, jnp.exp(o) - 1.0)
    h2 = jnp.dot(o, w2_ref[...], preferred_element_type=_f32)
    a2 = jnp.dot(h2, as2_ref[...], preferred_element_type=_f32)
    b2 = jnp.dot(h2, bs2_ref[...], preferred_element_type=_f32)
    h2_ref[...] = jnp.concatenate([h2, a2], axis=1)
    b2_ref[...] = b2
    am = jnp.max(a2, axis=0, keepdims=True)
    bm = jnp.max(b2, axis=0, keepdims=True)

    @pl.when(i == 0)
    def _():
        ca_ref[...] = am
        cb_ref[...] = bm

    @pl.when(i != 0)
    def _():
        ca_ref[...] = jnp.maximum(ca_ref[...], am)
        cb_ref[...] = jnp.maximum(cb_ref[...], bm)


def _k3_body(acc_ref, bias_ref, g_ref, be_ref, o_ref):
    acc = acc_ref[...]
    s = acc[0] + acc[1]
    num = s[:, :40]
    den = s[:, 48:49]
    o = num / (den + 1e-16) + bias_ref[...]
    mu = jnp.mean(o, axis=-1, keepdims=True)
    var = jnp.mean((o - mu) ** 2, axis=-1, keepdims=True)
    o_ref[...] = (o - mu) / jnp.sqrt(var + 1e-5) * g_ref[...] + be_ref[...]


def _full(shape):
    return pl.BlockSpec(shape, lambda i: tuple(0 for _ in shape))


@jax.jit
def kernel(x, edge_index, W1, att_src1, att_dst1, bias1, gamma1, beta1, W2,
           att_src2, att_dst2, bias2, gamma2, beta2):
    src, dst = edge_index[0], edge_index[1]
    loop = jnp.arange(_N, dtype=src.dtype)
    padn = _EALLOC - _ETOT
    srcp = jnp.concatenate([src, loop, jnp.zeros((padn,), src.dtype)])
    dstp = jnp.concatenate([dst, loop, jnp.full((padn,), _TRASH, dst.dtype)])

    eye8 = jnp.eye(8, dtype=_f32)
    ms = (eye8[:, None, :] * att_src1[:, :, None]).reshape(64, 8)
    ms = jnp.concatenate([ms, jnp.zeros((64, 8), _f32)], axis=1)
    md = (eye8[:, None, :] * att_dst1[:, :, None]).reshape(64, 8)
    md = jnp.concatenate([md, jnp.zeros((64, 8), _f32)], axis=1)
    kb = jnp.kron(eye8, jnp.ones((1, 8), _f32))
    w2p = jnp.concatenate([W2, jnp.zeros((64, 8), _f32)], axis=1)
    as2 = jnp.concatenate(
        [jnp.tile(att_src2.reshape(40, 1), (1, 16)), jnp.zeros((8, 16), _f32)])
    bs2 = jnp.concatenate(
        [jnp.tile(att_dst2.reshape(40, 1), (1, 16)), jnp.zeros((8, 16), _f32)])

    grid = _N // _BN
    h1a, b1, ca1, cb1 = pl.pallas_call(
        _k1_body,
        grid=(grid,),
        in_specs=[
            pl.BlockSpec((_BN, 128), lambda i: (i, 0)),
            _full((128, 64)),
            _full((64, 16)),
            _full((64, 16)),
        ],
        out_specs=[
            pl.BlockSpec((_BN, 80), lambda i: (i, 0)),
            pl.BlockSpec((_BN, 16), lambda i: (i, 0)),
            _full((1, 16)),
            _full((1, 16)),
        ],
        out_shape=[
            jax.ShapeDtypeStruct((_N, 80), _f32),
            jax.ShapeDtypeStruct((_N, 16), _f32),
            jax.ShapeDtypeStruct((1, 16), _f32),
            jax.ShapeDtypeStruct((1, 16), _f32),
        ],
    )(x, W1, ms, md)

    c1 = jax.nn.leaky_relu(ca1 + cb1, 0.2).reshape(16)
    b1p = jnp.concatenate([b1, jnp.zeros((_NPAD - _N, 16), _f32)], axis=0)
    z1 = jnp.zeros((_NPAD, 80), _f32)
    acc1 = _edge1(h1a, b1p, srcp, dstp, c1, z1)

    h2a, b2, ca2, cb2 = pl.pallas_call(
        _k2_body,
        grid=(grid,),
        in_specs=[
            pl.BlockSpec((_NC, _BN, 80), lambda i: (0, i, 0)),
            _full((8, 64)),
            _full((1, 64)),
            _full((1, 64)),
            _full((1, 64)),
            _full((64, 48)),
            _full((48, 16)),
            _full((48, 16)),
        ],
        out_specs=[
            pl.BlockSpec((_BN, 64), lambda i: (i, 0)),
            pl.BlockSpec((_BN, 16), lambda i: (i, 0)),
            _full((1, 16)),
            _full((1, 16)),
        ],
        out_shape=[
            jax.ShapeDtypeStruct((_N, 64), _f32),
            jax.ShapeDtypeStruct((_N, 16), _f32),
            jax.ShapeDtypeStruct((1, 16), _f32),
            jax.ShapeDtypeStruct((1, 16), _f32),
        ],
    )(acc1, kb, bias1.reshape(1, 64), gamma1.reshape(1, 64),
      beta1.reshape(1, 64), w2p, as2, bs2)

    c2 = jax.nn.leaky_relu(ca2 + cb2, 0.2).reshape(16)
    b2p = jnp.concatenate([b2, jnp.zeros((_NPAD - _N, 16), _f32)], axis=0)
    z2 = jnp.zeros((_NPAD, 64), _f32)
    acc2 = _edge2(h2a, b2p, srcp, dstp, c2, z2)

    out = pl.pallas_call(
        _k3_body,
        grid=(grid,),
        in_specs=[
            pl.BlockSpec((_NC, _BN, 64), lambda i: (0, i, 0)),
            _full((1, 40)),
            _full((1, 40)),
            _full((1, 40)),
        ],
        out_specs=pl.BlockSpec((_BN, 40), lambda i: (i, 0)),
        out_shape=jax.ShapeDtypeStruct((_N, 40), _f32),
    )(acc2, bias2.reshape(1, 40), gamma2.reshape(1, 40),
      beta2.reshape(1, 40))
    return out


# speedup vs baseline: 2.4862x; 2.4862x over previous
import functools

import jax
import jax.numpy as jnp
from jax import lax
from jax.experimental import pallas as pl
from jax.experimental.pallas import tpu as pltpu
from jax.experimental.pallas import tpu_sc as plsc

_N = 10000
_E = 320000
_NPAD = 10112
_TRASH = _N
_NC, _NS = 2, 16
_NTILES = _NC * _NS
_BE = 144
_BH = 72
_NB = 3
_ETOT = _E + _N
_KCH = _NB * (-(-_ETOT // (_NTILES * _BE * _NB)))
_EPAD = _NTILES * _BE * _KCH
_EALLOC = _EPAD + _NB * _BE
_BN = 1000

_f32 = jnp.float32
_i32 = jnp.int32

_GDN = lax.GatherDimensionNumbers(
    offset_dims=(), collapsed_slice_dims=(0,), start_index_map=(0,))


def _vgather(vec, idx):
    return lax.gather(vec, idx[:, None], _GDN, (1,),
                      mode=lax.GatherScatterMode.PROMISE_IN_BOUNDS)


def _make_edge_kernel(srow, excol, nck, per_head):
    mesh = plsc.VectorSubcoreMesh(
        core_axis_name="c", subcore_axis_name="s",
        num_cores=_NC, num_subcores=_NS)
    rows_sub = _NPAD // _NS

    @functools.partial(
        pl.kernel,
        out_type=jax.ShapeDtypeStruct((_NC, _NPAD, srow), _f32),
        mesh=mesh,
        compiler_params=pltpu.CompilerParams(
            use_tc_tiling_on_sc=False, needs_layout_passes=False),
        scratch_types=[
            [pltpu.VMEM((_BE,), _i32)] * _NB,
            [pltpu.VMEM((_BE,), _i32)] * _NB,
            [[pltpu.VMEM((_BH,), _i32)] * 2] * _NB,
            [pltpu.VMEM((_BE, srow), _f32)] * _NB,
            [pltpu.VMEM((_BE, 16), _f32)] * _NB,
            [pltpu.VMEM((_BE, srow), _f32)] * _NB,
            pltpu.VMEM((16,), _f32),
            pltpu.VMEM_SHARED((_NPAD, srow), _f32),
            [pltpu.SemaphoreType.DMA] * _NB,
            [pltpu.SemaphoreType.DMA] * _NB,
            [pltpu.SemaphoreType.DMA] * _NB,
        ],
    )
    def edge_kernel(h_hbm, b_hbm, src_hbm, dst_hbm, cvec_hbm, zeros_hbm,
                    out_hbm, sidx, didx, didxs, hrows, brows, scat,
                    cvec, acc, sem_g, sem_s, sem_i):
        cid = lax.axis_index("c")
        sid = lax.axis_index("s")
        wid = cid * _NS + sid
        rb = sid * rows_sub
        pltpu.sync_copy(zeros_hbm.at[pl.ds(rb, rows_sub)],
                        acc.at[pl.ds(rb, rows_sub)])
        pltpu.sync_copy(cvec_hbm, cvec)
        plsc.subcore_barrier()
        cv = cvec[...]
        ii = lax.iota(_i32, 16)
        half = lax.shift_right_logical(ii, 3)
        scale_idx = [half + 2 * k for k in range(nck)]
        ebase = wid * (_KCH * _BE)

        def load_idx(cb, b):
            pltpu.sync_copy(src_hbm.at[pl.ds(cb, _BE)], sidx[b])
            pltpu.sync_copy(dst_hbm.at[pl.ds(cb, _BE)], didx[b])

        def _gather_parts(b):
            out = []
            for j in range(2):
                hs = pl.ds(j * _BH, _BH)
                out.append((h_hbm.at[sidx[b].at[hs]], hrows[b].at[hs]))
                out.append((b_hbm.at[didx[b].at[hs]], brows[b].at[hs]))
            return out

        def issue_gathers(b):
            for src, dst in _gather_parts(b):
                pltpu.async_copy(src, dst, sem_g[b])

        def wait_gathers(b):
            for src, dst in _gather_parts(b):
                pltpu.make_async_copy(src, dst, sem_g[b]).wait()

        def issue_scatters(b):
            for j in range(2):
                pltpu.async_copy(scat[b].at[pl.ds(j * _BH, _BH)],
                                 acc.at[didxs[b][j]], sem_s[b], add=True)

        def wait_scatter(b):
            for j in range(2):
                pltpu.make_async_copy(scat[b].at[pl.ds(j * _BH, _BH)],
                                      acc.at[didxs[b][j]], sem_s[b]).wait()

        def compute(b):
            hr = hrows[b]
            br = brows[b]
            st = scat[b]

            @plsc.parallel_loop(0, _BE, unroll=16)
            def _(e):
                z = hr[e, pl.ds(excol, 16)] + br[e, :]
                zz = jnp.where(z >= 0, z, 0.2 * z) - cv
                ex = jnp.exp(zz)
                for k2 in range(nck):
                    sc = _vgather(ex, scale_idx[k2]) if per_head else ex
                    hv = hr[e, pl.ds(k2 * 16, 16)]
                    st[e, pl.ds(k2 * 16, 16)] = hv * sc
                st[e, pl.ds(excol, 16)] = ex

        for b in range(_NB):
            load_idx(ebase + b * _BE, b)
            issue_gathers(b)

        def ringbody(go, carry):
            for b in range(_NB):
                wait_gathers(b)

                @pl.when(go > 0)
                def _():
                    wait_scatter(b)

                cb0 = ebase + (go * _NB + b) * _BE
                cp_i0 = pltpu.async_copy(dst_hbm.at[pl.ds(cb0, _BH)],
                                         didxs[b][0], sem_i[b])
                cp_i0b = pltpu.async_copy(dst_hbm.at[pl.ds(cb0 + _BH, _BH)],
                                          didxs[b][1], sem_i[b])
                cb2 = cb0 + _NB * _BE
                cp_i1 = pltpu.async_copy(src_hbm.at[pl.ds(cb2, _BE)],
                                         sidx[b], sem_i[b])
                cp_i2 = pltpu.async_copy(dst_hbm.at[pl.ds(cb2, _BE)],
                                         didx[b], sem_i[b])
                compute(b)
                cp_i0.wait()
                cp_i0b.wait()
                cp_i1.wait()
                cp_i2.wait()
                issue_scatters(b)
                issue_gathers(b)
            return carry

        lax.fori_loop(0, _KCH // _NB, ringbody, 0)
        for b in range(_NB):
            wait_gathers(b)
            wait_scatter(b)
        plsc.subcore_barrier()
        pltpu.sync_copy(acc.at[pl.ds(rb, rows_sub)],
                        out_hbm.at[cid, pl.ds(rb, rows_sub)])

    return edge_kernel


_edge1 = _make_edge_kernel(srow=80, excol=64, nck=4, per_head=True)
_edge2 = _make_edge_kernel(srow=64, excol=48, nck=3, per_head=False)


def _k1_body(x_ref, w1_ref, ms_ref, md_ref, ha_ref, b_ref, ca_ref,
             cb_ref):
    i = pl.program_id(0)
    h = jnp.dot(x_ref[...], w1_ref[...], preferred_element_type=_f32)
    a = jnp.dot(h, ms_ref[...], preferred_element_type=_f32)
    b = jnp.dot(h, md_ref[...], preferred_element_type=_f32)
    ha_ref[...] = jnp.concatenate([h, a], axis=1)
    b_ref[...] = b
    am = jnp.max(a, axis=0, keepdims=True)
    bm = jnp.max(b, axis=0, keepdims=True)

    @pl.when(i == 0)
    def _():
        ca_ref[...] = am
        cb_ref[...] = bm

    @pl.when(i != 0)
    def _():
        ca_ref[...] = jnp.maximum(ca_ref[...], am)
        cb_ref[...] = jnp.maximum(cb_ref[...], bm)


def _k2_body(acc_ref, kb_ref, bias_ref, g_ref, be_ref, w2_ref, as2_ref,
             bs2_ref, h2_ref, b2_ref, ca_ref, cb_ref):
    i = pl.program_id(0)
    acc = acc_ref[...]
    s = acc[0] + acc[1]
    num = s[:, :64]
    den = s[:, 64:72]
    denf = jnp.dot(den, kb_ref[...], preferred_element_type=_f32)
    o = num / (denf + 1e-16) + bias_ref[...]
    mu = jnp.mean(o, axis=-1, keepdims=True)
    var = jnp.mean((o - mu) ** 2, axis=-1, keepdims=True)
    o = (o - mu) / jnp.sqrt(var + 1e-5) * g_ref[...] + be_ref[...]
    o = jnp.where(o > 0, o, jnp.exp(o) - 1.0)
    h2 = jnp.dot(o, w2_ref[...], preferred_element_type=_f32)
    a2 = jnp.dot(h2, as2_ref[...], preferred_element_type=_f32)
    b2 = jnp.dot(h2, bs2_ref[...], preferred_element_type=_f32)
    h2_ref[...] = jnp.concatenate([h2, a2], axis=1)
    b2_ref[...] = b2
    am = jnp.max(a2, axis=0, keepdims=True)
    bm = jnp.max(b2, axis=0, keepdims=True)

    @pl.when(i == 0)
    def _():
        ca_ref[...] = am
        cb_ref[...] = bm

    @pl.when(i != 0)
    def _():
        ca_ref[...] = jnp.maximum(ca_ref[...], am)
        cb_ref[...] = jnp.maximum(cb_ref[...], bm)


def _k3_body(acc_ref, bias_ref, g_ref, be_ref, o_ref):
    acc = acc_ref[...]
    s = acc[0] + acc[1]
    num = s[:, :40]
    den = s[:, 48:49]
    o = num / (den + 1e-16) + bias_ref[...]
    mu = jnp.mean(o, axis=-1, keepdims=True)
    var = jnp.mean((o - mu) ** 2, axis=-1, keepdims=True)
    o_ref[...] = (o - mu) / jnp.sqrt(var + 1e-5) * g_ref[...] + be_ref[...]


def _full(shape):
    return pl.BlockSpec(shape, lambda i: tuple(0 for _ in shape))


@jax.jit
def kernel(x, edge_index, W1, att_src1, att_dst1, bias1, gamma1, beta1, W2,
           att_src2, att_dst2, bias2, gamma2, beta2):
    src, dst = edge_index[0], edge_index[1]
    loop = jnp.arange(_N, dtype=src.dtype)
    padn = _EALLOC - _ETOT
    srcp = jnp.concatenate([src, loop, jnp.zeros((padn,), src.dtype)])
    dstp = jnp.concatenate([dst, loop, jnp.full((padn,), _TRASH, dst.dtype)])

    eye8 = jnp.eye(8, dtype=_f32)
    ms = (eye8[:, None, :] * att_src1[:, :, None]).reshape(64, 8)
    ms = jnp.concatenate([ms, jnp.zeros((64, 8), _f32)], axis=1)
    md = (eye8[:, None, :] * att_dst1[:, :, None]).reshape(64, 8)
    md = jnp.concatenate([md, jnp.zeros((64, 8), _f32)], axis=1)
    kb = jnp.kron(eye8, jnp.ones((1, 8), _f32))
    w2p = jnp.concatenate([W2, jnp.zeros((64, 8), _f32)], axis=1)
    as2 = jnp.concatenate(
        [jnp.tile(att_src2.reshape(40, 1), (1, 16)), jnp.zeros((8, 16), _f32)])
    bs2 = jnp.concatenate(
        [jnp.tile(att_dst2.reshape(40, 1), (1, 16)), jnp.zeros((8, 16), _f32)])

    grid = _N // _BN
    h1a, b1, ca1, cb1 = pl.pallas_call(
        _k1_body,
        grid=(grid,),
        in_specs=[
            pl.BlockSpec((_BN, 128), lambda i: (i, 0)),
            _full((128, 64)),
            _full((64, 16)),
            _full((64, 16)),
        ],
        out_specs=[
            pl.BlockSpec((_BN, 80), lambda i: (i, 0)),
            pl.BlockSpec((_BN, 16), lambda i: (i, 0)),
            _full((1, 16)),
            _full((1, 16)),
        ],
        out_shape=[
            jax.ShapeDtypeStruct((_N, 80), _f32),
            jax.ShapeDtypeStruct((_N, 16), _f32),
            jax.ShapeDtypeStruct((1, 16), _f32),
            jax.ShapeDtypeStruct((1, 16), _f32),
        ],
    )(x, W1, ms, md)

    c1 = jax.nn.leaky_relu(ca1 + cb1, 0.2).reshape(16)
    b1p = jnp.concatenate([b1, jnp.zeros((_NPAD - _N, 16), _f32)], axis=0)
    z1 = jnp.zeros((_NPAD, 80), _f32)
    acc1 = _edge1(h1a, b1p, srcp, dstp, c1, z1)

    h2a, b2, ca2, cb2 = pl.pallas_call(
        _k2_body,
        grid=(grid,),
        in_specs=[
            pl.BlockSpec((_NC, _BN, 80), lambda i: (0, i, 0)),
            _full((8, 64)),
            _full((1, 64)),
            _full((1, 64)),
            _full((1, 64)),
            _full((64, 48)),
            _full((48, 16)),
            _full((48, 16)),
        ],
        out_specs=[
            pl.BlockSpec((_BN, 64), lambda i: (i, 0)),
            pl.BlockSpec((_BN, 16), lambda i: (i, 0)),
            _full((1, 16)),
            _full((1, 16)),
        ],
        out_shape=[
            jax.ShapeDtypeStruct((_N, 64), _f32),
            jax.ShapeDtypeStruct((_N, 16), _f32),
            jax.ShapeDtypeStruct((1, 16), _f32),
            jax.ShapeDtypeStruct((1, 16), _f32),
        ],
    )(acc1, kb, bias1.reshape(1, 64), gamma1.reshape(1, 64),
      beta1.reshape(1, 64), w2p, as2, bs2)

    c2 = jax.nn.leaky_relu(ca2 + cb2, 0.2).reshape(16)
    b2p = jnp.concatenate([b2, jnp.zeros((_NPAD - _N, 16), _f32)], axis=0)
    z2 = jnp.zeros((_NPAD, 64), _f32)
    acc2 = _edge2(h2a, b2p, srcp, dstp, c2, z2)

    out = pl.pallas_call(
        _k3_body,
        grid=(grid,),
        in_specs=[
            pl.BlockSpec((_NC, _BN, 64), lambda i: (0, i, 0)),
            _full((1, 40)),
            _full((1, 40)),
            _full((1, 40)),
        ],
        out_specs=pl.BlockSpec((_BN, 40), lambda i: (i, 0)),
        out_shape=jax.ShapeDtypeStruct((_N, 40), _f32),
    )(acc2, bias2.reshape(1, 40), gamma2.reshape(1, 40),
      beta2.reshape(1, 40))
    return out
